# baseline (speedup 1.0000x reference)
import functools

import jax
import jax.numpy as jnp
from jax import lax
from jax.experimental import pallas as pl
from jax.experimental.pallas import tpu as pltpu
from jax.experimental.pallas import tpu_sc as plsc

N = 4096
B = 2
BN = 512
NB = N // BN
DP = 8
CH = 1024
NV = CH // 16


def _argmin_body(q_ref, qt_ref, k_ref, kt_ref, o_ref, kk_ref):
    j = pl.program_id(1)

    @pl.when(j == 0)
    def _():
        kt = kt_ref[0, 0]
        kk_ref[0, :] = jnp.sum(kt * kt, axis=0)

    q = q_ref[0, 0]
    k = k_ref[0, 0]
    qt = qt_ref[0, 0]
    qq = jnp.sum(qt * qt, axis=0)
    qkn = lax.dot_general(-2.0 * q, k, (((1,), (1,)), ((), ())),
                          preferred_element_type=jnp.float32)
    d2 = qq[:, None] + qkn + kk_ref[0, :][None, :]
    o_ref[0, 0, :] = jnp.argmin(d2, axis=1).astype(jnp.int32)


_argmin_call = pl.pallas_call(
    _argmin_body,
    grid=(4 * B, NB),
    in_specs=[
        pl.BlockSpec((1, 1, BN, 3), lambda i, j: (i // B, i % B, j, 0)),
        pl.BlockSpec((1, 1, 3, BN), lambda i, j: (i // B, i % B, 0, j)),
        pl.BlockSpec((1, 1, N, 3), lambda i, j: ((i // B) ^ 1, i % B, 0, 0)),
        pl.BlockSpec((1, 1, 3, N), lambda i, j: ((i // B) ^ 1, i % B, 0, 0)),
    ],
    out_specs=pl.BlockSpec((1, 1, BN), lambda i, j: (i, 0, j)),
    out_shape=jax.ShapeDtypeStruct((4 * B, 1, N), jnp.int32),
    scratch_shapes=[pltpu.VMEM((1, N), jnp.float32)],
    compiler_params=pltpu.CompilerParams(
        dimension_semantics=("parallel", "arbitrary")),
)


def _sc_body(pts, nn, li, out,
             gxyz, dxyz, gx, gy, gz, dx, dy, dz, ixr, ox, oy, oz, pv, psr,
             sh_th, sh_ps):
    cid = lax.axis_index("c")
    sid = lax.axis_index("s")
    a = sid & 3
    sub = sid >> 2
    c0 = sub * CH

    def prow(s):
        return (s * B + cid) * N * 3

    def shplane(s, d):
        return (s * 3 + d) * N

    gsrc = a ^ 1
    pltpu.sync_copy(pts.at[pl.ds(prow(gsrc), N * 3)], gxyz)
    pltpu.sync_copy(pts.at[pl.ds(prow(a) + c0 * 3, CH * 3)], dxyz)
    pltpu.sync_copy(nn.at[pl.ds((a * B + cid) * N + c0, CH)], ixr)

    sgn = jnp.where(a & 1 == 0, jnp.float32(1.0), jnp.float32(-1.0))
    sgv = jnp.full((16,), sgn, jnp.float32)
    l3 = lax.iota(jnp.int32, 16) * 3

    def body1(i, _):
        i3 = ixr[pl.ds(i * 16, 16)] * 3
        d3 = l3 + i * 48
        for d, o in ((0, ox), (1, oy), (2, oz)):
            gv = plsc.load_gather(gxyz, [i3 + d])
            dv = plsc.load_gather(dxyz, [d3 + d])
            o[pl.ds(i * 16, 16)] = sgv * (gv - dv)
        return 0

    lax.fori_loop(0, NV, body1, 0)
    pltpu.sync_copy(ox, sh_th.at[pl.ds(shplane(a, 0) + c0, CH)])
    pltpu.sync_copy(oy, sh_th.at[pl.ds(shplane(a, 1) + c0, CH)])
    pltpu.sync_copy(oz, sh_th.at[pl.ds(shplane(a, 2) + c0, CH)])

    plsc.subcore_barrier()

    darr = ((a & 1) << 1) | (a >> 1)
    garr = darr ^ 2
    pltpu.sync_copy(sh_th.at[pl.ds(shplane(garr, 0), N)], gx)
    pltpu.sync_copy(sh_th.at[pl.ds(shplane(garr, 1), N)], gy)
    pltpu.sync_copy(sh_th.at[pl.ds(shplane(garr, 2), N)], gz)
    pltpu.sync_copy(sh_th.at[pl.ds(shplane(darr, 0) + c0, CH)], dx)
    pltpu.sync_copy(sh_th.at[pl.ds(shplane(darr, 1) + c0, CH)], dy)
    pltpu.sync_copy(sh_th.at[pl.ds(shplane(darr, 2) + c0, CH)], dz)
    pltpu.sync_copy(li.at[pl.ds((a * B + cid) * N + c0, CH)], ixr)

    def body2(i, acc):
        idx = ixr[pl.ds(i * 16, 16)]
        for g, d in ((gx, dx), (gy, dy), (gz, dz)):
            gv = plsc.load_gather(g, [idx])
            dv = d[pl.ds(i * 16, 16)]
            df = dv - gv
            acc = acc + df * df
        return acc

    acc = lax.fori_loop(0, NV, body2, jnp.zeros((16,), jnp.float32))
    pv[...] = acc
    pltpu.sync_copy(pv, sh_ps.at[pl.ds(sid * 16, 16)])

    plsc.subcore_barrier()

    @pl.when(sid == 0)
    def _():
        pltpu.sync_copy(sh_ps, psr)
        tot = jnp.zeros((16,), jnp.float32)
        for i in range(16):
            tot = tot + psr[pl.ds(i * 16, 16)]
        s = jnp.sum(tot) * jnp.float32(0.25 / (B * N))
        pv[...] = jnp.full((16,), s, jnp.float32)
        pltpu.sync_copy(pv, out.at[pl.ds(cid * 16, 16)])


@functools.lru_cache(maxsize=None)
def _sc_call():
  return functools.partial(
    pl.kernel,
    mesh=plsc.VectorSubcoreMesh(core_axis_name="c", subcore_axis_name="s"),
    out_type=jax.ShapeDtypeStruct((2 * 16,), jnp.float32),
    compiler_params=pltpu.CompilerParams(needs_layout_passes=False),
    scratch_types=[
        pltpu.VMEM((N * 3,), jnp.float32),
        pltpu.VMEM((CH * 3,), jnp.float32),
        pltpu.VMEM((N,), jnp.float32),
        pltpu.VMEM((N,), jnp.float32),
        pltpu.VMEM((N,), jnp.float32),
        pltpu.VMEM((CH,), jnp.float32),
        pltpu.VMEM((CH,), jnp.float32),
        pltpu.VMEM((CH,), jnp.float32),
        pltpu.VMEM((CH,), jnp.int32),
        pltpu.VMEM((CH,), jnp.float32),
        pltpu.VMEM((CH,), jnp.float32),
        pltpu.VMEM((CH,), jnp.float32),
        pltpu.VMEM((16,), jnp.float32),
        pltpu.VMEM((16 * 16,), jnp.float32),
        pltpu.VMEM_SHARED((4 * 3 * N,), jnp.float32),
        pltpu.VMEM_SHARED((16 * 16,), jnp.float32),
    ],
  )(_sc_body)


@jax.jit
def kernel(yp_white_pts, yp_pial_pts, yt_white_pts, yt_pial_pts,
           yp_white_idx, yt_white_idx, yp_pial_idx, yt_pial_idx):
    pts4 = jnp.stack([yp_white_pts, yp_pial_pts,
                      yt_white_pts, yt_pial_pts])
    pts4t = jnp.transpose(pts4, (0, 1, 3, 2))

    nn_idx = _argmin_call(pts4, pts4t, pts4, pts4t)

    pts_flat = pts4.reshape(-1)
    nn_flat = nn_idx.reshape(-1)
    li_flat = jnp.stack([yp_white_idx, yt_white_idx,
                         yp_pial_idx, yt_pial_idx]).astype(jnp.int32).reshape(-1)

    out = _sc_call()(pts_flat, nn_flat, li_flat)
    return out[0] + out[16]


# speedup vs baseline: 1.0609x; 1.0609x over previous
import functools

import jax
import jax.numpy as jnp
from jax import lax
from jax.experimental import pallas as pl
from jax.experimental.pallas import tpu as pltpu
from jax.experimental.pallas import tpu_sc as plsc

N = 4096
B = 2
BN = 512
NB = N // BN
DP = 8
CH = 1024
NV = CH // 16


def _argmin_body(q_ref, qt_ref, k_ref, kt_ref, o_ref, kk_ref):
    j = pl.program_id(1)

    @pl.when(j == 0)
    def _():
        kt = kt_ref[0, 0]
        kk_ref[0, :] = jnp.sum(kt * kt, axis=0)

    q = q_ref[0, 0]
    k = k_ref[0, 0]
    qt = qt_ref[0, 0]
    qq = jnp.sum(qt * qt, axis=0)
    qkn = lax.dot_general(-2.0 * q, k, (((1,), (1,)), ((), ())),
                          preferred_element_type=jnp.float32)
    d2 = qq[:, None] + qkn + kk_ref[0, :][None, :]
    o_ref[0, 0, :] = jnp.argmin(d2, axis=1).astype(jnp.int32)


_argmin_call = pl.pallas_call(
    _argmin_body,
    grid=(4 * B, NB),
    in_specs=[
        pl.BlockSpec((1, 1, BN, 3), lambda i, j: (i // B, i % B, j, 0)),
        pl.BlockSpec((1, 1, 3, BN), lambda i, j: (i // B, i % B, 0, j)),
        pl.BlockSpec((1, 1, N, 3), lambda i, j: ((i // B) ^ 1, i % B, 0, 0)),
        pl.BlockSpec((1, 1, 3, N), lambda i, j: ((i // B) ^ 1, i % B, 0, 0)),
    ],
    out_specs=pl.BlockSpec((1, 1, BN), lambda i, j: (i, 0, j)),
    out_shape=jax.ShapeDtypeStruct((4 * B, 1, N), jnp.int32),
    scratch_shapes=[pltpu.VMEM((1, N), jnp.float32)],
    compiler_params=pltpu.CompilerParams(
        dimension_semantics=("parallel", "arbitrary")),
)


def _sc_body(pts, nn, li, out,
             gx, gy, gz, dx, dy, dz, ixr, ox, oy, oz, pv, psr,
             sh_th, sh_ps):
    cid = lax.axis_index("c")
    sid = lax.axis_index("s")
    a = sid & 3
    sub = sid >> 2
    c0 = sub * CH

    def plane(s, d):
        return ((s * B + cid) * 3 + d) * N

    def shplane(s, d):
        return (s * 3 + d) * N

    gsrc = a ^ 1
    pltpu.sync_copy(pts.at[pl.ds(plane(gsrc, 0), N)], gx)
    pltpu.sync_copy(pts.at[pl.ds(plane(gsrc, 1), N)], gy)
    pltpu.sync_copy(pts.at[pl.ds(plane(gsrc, 2), N)], gz)
    pltpu.sync_copy(pts.at[pl.ds(plane(a, 0) + c0, CH)], dx)
    pltpu.sync_copy(pts.at[pl.ds(plane(a, 1) + c0, CH)], dy)
    pltpu.sync_copy(pts.at[pl.ds(plane(a, 2) + c0, CH)], dz)
    pltpu.sync_copy(nn.at[pl.ds((a * B + cid) * N + c0, CH)], ixr)

    sgn = jnp.where(a & 1 == 0, jnp.float32(1.0), jnp.float32(-1.0))
    sgv = jnp.full((16,), sgn, jnp.float32)

    def body1(i, _):
        idx = ixr[pl.ds(i * 16, 16)]
        for g, d, o in ((gx, dx, ox), (gy, dy, oy), (gz, dz, oz)):
            gv = plsc.load_gather(g, [idx])
            dv = d[pl.ds(i * 16, 16)]
            o[pl.ds(i * 16, 16)] = sgv * (gv - dv)
        return 0

    lax.fori_loop(0, NV, body1, 0)
    pltpu.sync_copy(ox, sh_th.at[pl.ds(shplane(a, 0) + c0, CH)])
    pltpu.sync_copy(oy, sh_th.at[pl.ds(shplane(a, 1) + c0, CH)])
    pltpu.sync_copy(oz, sh_th.at[pl.ds(shplane(a, 2) + c0, CH)])

    plsc.subcore_barrier()

    darr = ((a & 1) << 1) | (a >> 1)
    garr = darr ^ 2
    pltpu.sync_copy(sh_th.at[pl.ds(shplane(garr, 0), N)], gx)
    pltpu.sync_copy(sh_th.at[pl.ds(shplane(garr, 1), N)], gy)
    pltpu.sync_copy(sh_th.at[pl.ds(shplane(garr, 2), N)], gz)
    pltpu.sync_copy(sh_th.at[pl.ds(shplane(darr, 0) + c0, CH)], dx)
    pltpu.sync_copy(sh_th.at[pl.ds(shplane(darr, 1) + c0, CH)], dy)
    pltpu.sync_copy(sh_th.at[pl.ds(shplane(darr, 2) + c0, CH)], dz)
    pltpu.sync_copy(li.at[pl.ds((a * B + cid) * N + c0, CH)], ixr)

    def body2(i, acc):
        idx = ixr[pl.ds(i * 16, 16)]
        for g, d in ((gx, dx), (gy, dy), (gz, dz)):
            gv = plsc.load_gather(g, [idx])
            dv = d[pl.ds(i * 16, 16)]
            df = dv - gv
            acc = acc + df * df
        return acc

    acc = lax.fori_loop(0, NV, body2, jnp.zeros((16,), jnp.float32))
    pv[...] = acc
    pltpu.sync_copy(pv, sh_ps.at[pl.ds(sid * 16, 16)])

    plsc.subcore_barrier()

    @pl.when(sid == 0)
    def _():
        pltpu.sync_copy(sh_ps, psr)
        tot = jnp.zeros((16,), jnp.float32)
        for i in range(16):
            tot = tot + psr[pl.ds(i * 16, 16)]
        s = jnp.sum(tot) * jnp.float32(0.25 / (B * N))
        pv[...] = jnp.full((16,), s, jnp.float32)
        pltpu.sync_copy(pv, out.at[pl.ds(cid * 16, 16)])


@functools.lru_cache(maxsize=None)
def _sc_call():
  return functools.partial(
    pl.kernel,
    mesh=plsc.VectorSubcoreMesh(core_axis_name="c", subcore_axis_name="s"),
    out_type=jax.ShapeDtypeStruct((2 * 16,), jnp.float32),
    compiler_params=pltpu.CompilerParams(needs_layout_passes=False),
    scratch_types=[
        pltpu.VMEM((N,), jnp.float32),
        pltpu.VMEM((N,), jnp.float32),
        pltpu.VMEM((N,), jnp.float32),
        pltpu.VMEM((CH,), jnp.float32),
        pltpu.VMEM((CH,), jnp.float32),
        pltpu.VMEM((CH,), jnp.float32),
        pltpu.VMEM((CH,), jnp.int32),
        pltpu.VMEM((CH,), jnp.float32),
        pltpu.VMEM((CH,), jnp.float32),
        pltpu.VMEM((CH,), jnp.float32),
        pltpu.VMEM((16,), jnp.float32),
        pltpu.VMEM((16 * 16,), jnp.float32),
        pltpu.VMEM_SHARED((4 * 3 * N,), jnp.float32),
        pltpu.VMEM_SHARED((16 * 16,), jnp.float32),
    ],
  )(_sc_body)


@jax.jit
def kernel(yp_white_pts, yp_pial_pts, yt_white_pts, yt_pial_pts,
           yp_white_idx, yt_white_idx, yp_pial_idx, yt_pial_idx):
    pts4 = jnp.stack([yp_white_pts, yp_pial_pts,
                      yt_white_pts, yt_pial_pts])
    pts4t = jnp.transpose(pts4, (0, 1, 3, 2))

    nn_idx = _argmin_call(pts4, pts4t, pts4, pts4t)

    pts_flat = pts4t.reshape(-1)
    nn_flat = nn_idx.reshape(-1)
    li_flat = jnp.stack([yp_white_idx, yt_white_idx,
                         yp_pial_idx, yt_pial_idx]).astype(jnp.int32).reshape(-1)

    out = _sc_call()(pts_flat, nn_flat, li_flat)
    return out[0] + out[16]
